# Initial kernel scaffold; baseline (speedup 1.0000x reference)
import jax
import jax.numpy as jnp
from jax.experimental import pallas as pl


def kernel(anchors, pred_objectness_logits, pred_anchor_deltas):
    raise NotImplementedError("write your pallas kernel here")


# speedup vs baseline: 3.3231x; 3.3231x over previous
import math

import jax
import jax.numpy as jnp
from jax.experimental import pallas as pl

_NMS_T = 0.7
_PRE = 2000
_POST = 1000
_PAD = 2048
_IMG = 1024.0
_CLAMP = math.log(1000.0 / 16.0)


def _rpn_body(anc_ref, dlt_ref, sc_ref, out_ref):
    f32 = jnp.float32
    rows, cols = 16, 128
    fi = (jax.lax.broadcasted_iota(jnp.int32, (rows, cols), 0) * cols
          + jax.lax.broadcasted_iota(jnp.int32, (rows, cols), 1))
    fi_f = fi.astype(f32)

    u128 = (jax.lax.broadcasted_iota(jnp.int32, (cols, cols), 0)
            <= jax.lax.broadcasted_iota(jnp.int32, (cols, cols), 1)).astype(f32)
    l16 = (jax.lax.broadcasted_iota(jnp.int32, (rows, rows), 0)
           > jax.lax.broadcasted_iota(jnp.int32, (rows, rows), 1)).astype(f32)
    ones_col = jnp.ones((cols, 1), f32)

    def excl_prefix(xm):
        incl = jnp.dot(xm, u128, preferred_element_type=f32)
        rt = jnp.dot(xm, ones_col, preferred_element_type=f32)
        offs = jnp.dot(l16, rt, preferred_element_type=f32)
        return (incl + offs) - xm

    for img in range(2):
        ax1 = anc_ref[img, 0]
        ay1 = anc_ref[img, 1]
        ax2 = anc_ref[img, 2]
        ay2 = anc_ref[img, 3]
        d0 = dlt_ref[img, 0]
        d1 = dlt_ref[img, 1]
        d2 = dlt_ref[img, 2]
        d3 = dlt_ref[img, 3]
        s = sc_ref[img]

        widths = ax2 - ax1
        heights = ay2 - ay1
        ctr_x = ax1 + 0.5 * widths
        ctr_y = ay1 + 0.5 * heights
        dw = jnp.minimum(d2, _CLAMP)
        dh = jnp.minimum(d3, _CLAMP)
        pred_ctr_x = d0 * widths + ctr_x
        pred_ctr_y = d1 * heights + ctr_y
        pred_w = jnp.exp(dw) * widths
        pred_h = jnp.exp(dh) * heights
        x1 = jnp.clip(pred_ctr_x - 0.5 * pred_w, 0.0, _IMG)
        y1 = jnp.clip(pred_ctr_y - 0.5 * pred_h, 0.0, _IMG)
        x2 = jnp.clip(pred_ctr_x + 0.5 * pred_w, 0.0, _IMG)
        y2 = jnp.clip(pred_ctr_y + 0.5 * pred_h, 0.0, _IMG)

        is_real = fi < _PRE
        valid = ((x2 - x1) > 0.0) & ((y2 - y1) > 0.0) & is_real
        s_eff = jnp.where(valid & is_real, s, jnp.where(is_real, -1e9, 0.0))
        areas = (x2 - x1) * (y2 - y1)

        def body(i, keep):
            m = fi == i
            k_i = jnp.sum(jnp.where(m, keep, 0.0))
            bx1 = jnp.sum(jnp.where(m, x1, 0.0))
            by1 = jnp.sum(jnp.where(m, y1, 0.0))
            bx2 = jnp.sum(jnp.where(m, x2, 0.0))
            by2 = jnp.sum(jnp.where(m, y2, 0.0))
            ba = jnp.sum(jnp.where(m, areas, 0.0))
            xx1 = jnp.maximum(bx1, x1)
            yy1 = jnp.maximum(by1, y1)
            xx2 = jnp.minimum(bx2, x2)
            yy2 = jnp.minimum(by2, y2)
            inter = jnp.maximum(xx2 - xx1, 0.0) * jnp.maximum(yy2 - yy1, 0.0)
            iou = inter / (ba + areas - inter + 1e-9)
            supp = (iou > _NMS_T) & (fi > i) & (k_i > 0.0)
            return jnp.where(supp, 0.0, keep)

        keep_f = jax.lax.fori_loop(0, _PRE, body, jnp.where(valid, 1.0, 0.0).astype(f32))
        keep = keep_f > 0.5

        kept_f = keep_f
        inv = (~valid) & is_real
        inv_f = jnp.where(inv, 1.0, 0.0).astype(f32)
        n_kept = jnp.sum(kept_f)
        r_kept = excl_prefix(kept_f)
        r_inv = excl_prefix(inv_f) + n_kept
        rank = jnp.where(keep, r_kept, jnp.where(inv, r_inv, 1e9))

        coords = (x1, y1, x2, y2, s_eff)
        for rb in range(32):
            base = rb * 32
            oi = (base + jax.lax.broadcasted_iota(jnp.int32, (32, 1, 1), 0)).astype(f32)
            oh = rank[None, :, :] == oi
            for c in range(5):
                chunk = jnp.sum(jnp.where(oh, coords[c][None, :, :], 0.0), axis=(1, 2))
                out_ref[img, c, pl.ds(base, 32)] = chunk


def kernel(anchors, pred_objectness_logits, pred_anchor_deltas):
    scores_k, idx = jax.lax.top_k(pred_objectness_logits, _PRE)
    anc_k = jnp.take(anchors, idx, axis=0)
    dlt_k = jnp.take_along_axis(pred_anchor_deltas, idx[..., None], axis=1)
    npad = _PAD - _PRE
    anc_p = jnp.pad(anc_k, ((0, 0), (0, npad), (0, 0)))
    dlt_p = jnp.pad(dlt_k, ((0, 0), (0, npad), (0, 0)))
    sc_p = jnp.pad(scores_k, ((0, 0), (0, npad)))
    anc_t = anc_p.transpose(0, 2, 1).reshape(2, 4, 16, 128)
    dlt_t = dlt_p.transpose(0, 2, 1).reshape(2, 4, 16, 128)
    sc_t = sc_p.reshape(2, 16, 128)
    out = pl.pallas_call(
        _rpn_body,
        out_shape=jax.ShapeDtypeStruct((2, 5, 1024), jnp.float32),
    )(anc_t, dlt_t, sc_t)
    return out.transpose(0, 2, 1)[:, :_POST, :]


# speedup vs baseline: 6.0367x; 1.8166x over previous
import math

import jax
import jax.numpy as jnp
from jax.experimental import pallas as pl

_NMS_T = 0.7
_PRE = 2000
_POST = 1000
_PAD = 2048
_IMG = 1024.0
_CLAMP = math.log(1000.0 / 16.0)


def _rpn_body(anc_ref, dlt_ref, sc_ref, out_ref):
    f32 = jnp.float32
    rows, cols = 16, 128
    fi = (jax.lax.broadcasted_iota(jnp.int32, (rows, cols), 0) * cols
          + jax.lax.broadcasted_iota(jnp.int32, (rows, cols), 1))

    u128 = (jax.lax.broadcasted_iota(jnp.int32, (cols, cols), 0)
            <= jax.lax.broadcasted_iota(jnp.int32, (cols, cols), 1)).astype(f32)
    l16 = (jax.lax.broadcasted_iota(jnp.int32, (rows, rows), 0)
           > jax.lax.broadcasted_iota(jnp.int32, (rows, rows), 1)).astype(f32)
    ones_col = jnp.ones((cols, 1), f32)

    def excl_prefix(xm):
        incl = jnp.dot(xm, u128, preferred_element_type=f32)
        rt = jnp.dot(xm, ones_col, preferred_element_type=f32)
        offs = jnp.dot(l16, rt, preferred_element_type=f32)
        return (incl + offs) - xm

    ax1 = anc_ref[:, 0]
    ay1 = anc_ref[:, 1]
    ax2 = anc_ref[:, 2]
    ay2 = anc_ref[:, 3]
    d0 = dlt_ref[:, 0]
    d1 = dlt_ref[:, 1]
    d2 = dlt_ref[:, 2]
    d3 = dlt_ref[:, 3]
    s = sc_ref[:]

    widths = ax2 - ax1
    heights = ay2 - ay1
    ctr_x = ax1 + 0.5 * widths
    ctr_y = ay1 + 0.5 * heights
    dw = jnp.minimum(d2, _CLAMP)
    dh = jnp.minimum(d3, _CLAMP)
    pred_ctr_x = d0 * widths + ctr_x
    pred_ctr_y = d1 * heights + ctr_y
    pred_w = jnp.exp(dw) * widths
    pred_h = jnp.exp(dh) * heights
    x1 = jnp.clip(pred_ctr_x - 0.5 * pred_w, 0.0, _IMG)
    y1 = jnp.clip(pred_ctr_y - 0.5 * pred_h, 0.0, _IMG)
    x2 = jnp.clip(pred_ctr_x + 0.5 * pred_w, 0.0, _IMG)
    y2 = jnp.clip(pred_ctr_y + 0.5 * pred_h, 0.0, _IMG)

    is_real = fi < _PRE
    valid = ((x2 - x1) > 0.0) & ((y2 - y1) > 0.0) & is_real[None]
    s_eff = jnp.where(valid, s, jnp.where(is_real[None], -1e9, 0.0))
    areas = (x2 - x1) * (y2 - y1)

    def body(i, keep):
        m = (fi == i)[None]
        ext = lambda v: jnp.sum(jnp.where(m, v, 0.0), axis=(1, 2), keepdims=True)
        k_i = ext(keep)
        bx1 = ext(x1)
        by1 = ext(y1)
        bx2 = ext(x2)
        by2 = ext(y2)
        ba = (bx2 - bx1) * (by2 - by1)
        xx1 = jnp.maximum(bx1, x1)
        yy1 = jnp.maximum(by1, y1)
        xx2 = jnp.minimum(bx2, x2)
        yy2 = jnp.minimum(by2, y2)
        inter = jnp.maximum(xx2 - xx1, 0.0) * jnp.maximum(yy2 - yy1, 0.0)
        iou = inter / (ba + areas - inter + 1e-9)
        supp = (iou > _NMS_T) & (fi > i)[None] & (k_i > 0.0)
        return jnp.where(supp, 0.0, keep)

    keep_f = jax.lax.fori_loop(0, _PRE, body, jnp.where(valid, 1.0, 0.0).astype(f32))
    keep = keep_f > 0.5

    for img in range(2):
        kept_f = keep_f[img]
        inv = (~valid[img]) & is_real
        inv_f = jnp.where(inv, 1.0, 0.0).astype(f32)
        n_kept = jnp.sum(kept_f)
        r_kept = excl_prefix(kept_f)
        r_inv = excl_prefix(inv_f) + n_kept
        rank = jnp.where(keep[img], r_kept, jnp.where(inv, r_inv, 1e9))

        coords = (x1[img], y1[img], x2[img], y2[img], s_eff[img])
        for rb in range(32):
            base = rb * 32
            oi = (base + jax.lax.broadcasted_iota(jnp.int32, (32, 1, 1), 0)).astype(f32)
            oh = rank[None, :, :] == oi
            for c in range(5):
                chunk = jnp.sum(jnp.where(oh, coords[c][None, :, :], 0.0), axis=(1, 2))
                out_ref[img, c, pl.ds(base, 32)] = chunk


def kernel(anchors, pred_objectness_logits, pred_anchor_deltas):
    scores_k, idx = jax.lax.top_k(pred_objectness_logits, _PRE)
    anc_k = jnp.take(anchors, idx, axis=0)
    dlt_k = jnp.take_along_axis(pred_anchor_deltas, idx[..., None], axis=1)
    npad = _PAD - _PRE
    anc_p = jnp.pad(anc_k, ((0, 0), (0, npad), (0, 0)))
    dlt_p = jnp.pad(dlt_k, ((0, 0), (0, npad), (0, 0)))
    sc_p = jnp.pad(scores_k, ((0, 0), (0, npad)))
    anc_t = anc_p.transpose(0, 2, 1).reshape(2, 4, 16, 128)
    dlt_t = dlt_p.transpose(0, 2, 1).reshape(2, 4, 16, 128)
    sc_t = sc_p.reshape(2, 16, 128)
    out = pl.pallas_call(
        _rpn_body,
        out_shape=jax.ShapeDtypeStruct((2, 5, 1024), jnp.float32),
    )(anc_t, dlt_t, sc_t)
    return out.transpose(0, 2, 1)[:, :_POST, :]
